# Initial kernel scaffold; baseline (speedup 1.0000x reference)
import jax
import jax.numpy as jnp
from jax.experimental import pallas as pl


def kernel(wav, wav_lens, phn, phn_lens):
    raise NotImplementedError("write your pallas kernel here")


# speedup vs baseline: 1.3624x; 1.3624x over previous
import math
from functools import lru_cache

import numpy as np
import jax
import jax.numpy as jnp
from jax.experimental import pallas as pl
from jax.experimental.pallas import tpu as pltpu

_SR = 16000
_NFFT = 321
_HOP = _NFFT // 2
_NFREQ = _NFFT // 2 + 1
_F2PAD = 384
_NMELS = 80
_MELPAD = 128
_TOPDB = 80.0
_NMFCC = 13
_DELTA_WIN = 5


def _rup(a, b):
    return -(-a // b) * b


@lru_cache(maxsize=8)
def _consts(t_frames):
    f64 = np.float64
    n = np.arange(_NFFT, dtype=f64)
    win = 0.5 - 0.5 * np.cos(2.0 * np.pi * n / _NFFT)
    k = np.arange(_NFREQ, dtype=f64)
    ang = 2.0 * np.pi * n[:, None] * k[None, :] / _NFFT
    basis = np.zeros((_NFFT, _F2PAD), f64)
    basis[:, :_NFREQ] = np.cos(ang) * win[:, None]
    basis[:, _NFREQ:2 * _NFREQ] = -np.sin(ang) * win[:, None]
    dmat = basis[:2 * _HOP]
    crow = basis[2 * _HOP:]

    hz2mel = lambda f: 2595.0 * np.log10(1.0 + f / 700.0)
    mel2hz = lambda m: 700.0 * (10.0 ** (m / 2595.0) - 1.0)
    freqs = np.linspace(0, _SR // 2, _NFREQ)
    mpts = mel2hz(np.linspace(hz2mel(0.0), hz2mel(_SR / 2.0), _NMELS + 2))
    fdiff = mpts[1:] - mpts[:-1]
    slopes = mpts[None, :] - freqs[:, None]
    fb = np.maximum(0.0, np.minimum(-slopes[:, :-2] / fdiff[:-1],
                                    slopes[:, 2:] / fdiff[1:]))
    fb2 = np.zeros((_F2PAD, _MELPAD), f64)
    fb2[:_NFREQ, :_NMELS] = fb
    fb2[_NFREQ:2 * _NFREQ, :_NMELS] = fb

    km = np.arange(_NMFCC, dtype=f64)[None, :]
    nm = np.arange(_NMELS, dtype=f64)[:, None]
    dct = np.cos(np.pi / _NMELS * (nm + 0.5) * km) * math.sqrt(2.0 / _NMELS)
    dct[:, 0] *= 1.0 / math.sqrt(2.0)
    dct_p = np.zeros((_MELPAD, _NMFCC), f64)
    dct_p[:_NMELS] = dct

    nd = (_DELTA_WIN - 1) // 2
    denom = nd * (nd + 1) * (2 * nd + 1) / 3.0
    dl = np.zeros((t_frames, t_frames), f64)
    ti = np.arange(t_frames)
    for j in range(-nd, nd + 1):
        np.add.at(dl, (ti, np.clip(ti + j, 0, t_frames - 1)), j / denom)

    scale = 0.5
    t_out = int(np.floor(t_frames * scale))
    src = np.maximum((np.arange(t_out) + 0.5) / scale - 0.5, 0.0)
    i0 = np.floor(src).astype(np.int64)
    i1 = np.minimum(i0 + 1, t_frames - 1)
    rs = np.zeros((t_out, t_frames), f64)
    np.add.at(rs, (np.arange(t_out), i0), 1.0 - (src - i0))
    np.add.at(rs, (np.arange(t_out), i1), src - i0)

    t_out_pad = max(8, _rup(t_out, 8))
    pstk = np.zeros((3 * t_out_pad, t_frames), f64)
    for g, m in enumerate((rs, rs @ dl, rs @ dl @ dl)):
        pstk[g * t_out_pad:g * t_out_pad + t_out] = m

    return (jnp.asarray(dmat, jnp.bfloat16), jnp.asarray(crow, jnp.float32),
            jnp.asarray(fb2, jnp.bfloat16), jnp.asarray(dct_p, jnp.float32),
            jnp.asarray(pstk, jnp.float32), t_out, t_out_pad)


def _make_spec_kernel(t_frames):
    def body(w_ref, rev_ref, dmat_ref, crow_ref, fb2_ref, mel_ref, max_ref):
        w = w_ref[0]
        w0 = jnp.concatenate([rev_ref[0], w[:t_frames - 1]],
                             axis=0).astype(jnp.bfloat16)
        w1 = w.astype(jnp.bfloat16)
        col = jnp.concatenate(
            [w[1:, 0:1], w[t_frames - 1:, _HOP - 2:_HOP - 1]], axis=0)

        y = jnp.dot(w0, dmat_ref[0:_HOP], preferred_element_type=jnp.float32)
        y = y + jnp.dot(w1, dmat_ref[_HOP:2 * _HOP],
                        preferred_element_type=jnp.float32)
        y = y + col * crow_ref[...]

        p = (y * y).astype(jnp.bfloat16)
        mel = jnp.dot(p, fb2_ref[...], preferred_element_type=jnp.float32)
        db = 10.0 * jnp.log10(jnp.maximum(mel, 1e-10))

        mel_ref[0] = db
        max_ref[0] = jnp.broadcast_to(jnp.max(db), (8, 128))
    return body


def _make_proj_kernel(t_out, t_out_pad):
    def body(max_ref, mel_ref, dct_ref, pstk_ref, out_ref):
        floor = jnp.max(max_ref[...]) - _TOPDB
        mel = jnp.maximum(mel_ref[0], floor)
        mfcc = jnp.dot(mel, dct_ref[...], preferred_element_type=jnp.float32)
        acc = jnp.dot(pstk_ref[...], mfcc, preferred_element_type=jnp.float32)
        out_ref[0] = jnp.concatenate(
            [acc[0:t_out],
             acc[t_out_pad:t_out_pad + t_out],
             acc[2 * t_out_pad:2 * t_out_pad + t_out]], axis=-1)
    return body


def kernel(wav, wav_lens, phn, phn_lens):
    del wav_lens, phn, phn_lens
    b = wav.shape[0]
    wav2d = wav.reshape(b, -1).astype(jnp.float32)
    L = wav2d.shape[1]
    assert L % _HOP == 0
    t = L // _HOP
    assert t % 8 == 0, "frame count must be a sublane multiple"

    dmat, crow, fb2, dct_p, pstk, t_out, t_out_pad = _consts(t)
    pr = pstk.shape[0]

    wb = wav2d.reshape(b, t, _HOP)
    rev = wav2d[:, 1:_HOP + 1][:, ::-1].reshape(b, 1, _HOP)

    mel_db, bmax = pl.pallas_call(
        _make_spec_kernel(t),
        out_shape=(
            jax.ShapeDtypeStruct((b, t, _MELPAD), jnp.float32),
            jax.ShapeDtypeStruct((b, 8, 128), jnp.float32),
        ),
        grid=(b,),
        in_specs=[
            pl.BlockSpec((1, t, _HOP), lambda i: (i, 0, 0)),
            pl.BlockSpec((1, 1, _HOP), lambda i: (i, 0, 0)),
            pl.BlockSpec((2 * _HOP, _F2PAD), lambda i: (0, 0)),
            pl.BlockSpec((1, _F2PAD), lambda i: (0, 0)),
            pl.BlockSpec((_F2PAD, _MELPAD), lambda i: (0, 0)),
        ],
        out_specs=[
            pl.BlockSpec((1, t, _MELPAD), lambda i: (i, 0, 0)),
            pl.BlockSpec((1, 8, 128), lambda i: (i, 0, 0)),
        ],
        compiler_params=pltpu.CompilerParams(
            dimension_semantics=("parallel",),
            vmem_limit_bytes=64 * 1024 * 1024,
        ),
    )(wb, rev, dmat, crow, fb2)

    out = pl.pallas_call(
        _make_proj_kernel(t_out, t_out_pad),
        out_shape=jax.ShapeDtypeStruct((b, t_out, 3 * _NMFCC), jnp.float32),
        grid=(b,),
        in_specs=[
            pl.BlockSpec((b, 8, 128), lambda i: (0, 0, 0)),
            pl.BlockSpec((1, t, _MELPAD), lambda i: (i, 0, 0)),
            pl.BlockSpec((_MELPAD, _NMFCC), lambda i: (0, 0)),
            pl.BlockSpec((pr, t), lambda i: (0, 0)),
        ],
        out_specs=pl.BlockSpec((1, t_out, 3 * _NMFCC), lambda i: (i, 0, 0)),
        compiler_params=pltpu.CompilerParams(
            dimension_semantics=("parallel",),
            vmem_limit_bytes=64 * 1024 * 1024,
        ),
    )(bmax, mel_db, dct_p, pstk)
    return [out]


# speedup vs baseline: 1.5913x; 1.1680x over previous
import math
from functools import lru_cache

import numpy as np
import jax
import jax.numpy as jnp
from jax.experimental import pallas as pl
from jax.experimental.pallas import tpu as pltpu

_SR = 16000
_NFFT = 321
_HOP = _NFFT // 2
_NFREQ = _NFFT // 2 + 1
_F2PAD = 384
_NMELS = 80
_MELPAD = 128
_TOPDB = 80.0
_NMFCC = 13
_DELTA_WIN = 5


def _rup(a, b):
    return -(-a // b) * b


@lru_cache(maxsize=8)
def _consts(t_frames):
    f64 = np.float64
    n = np.arange(_NFFT, dtype=f64)
    win = 0.5 - 0.5 * np.cos(2.0 * np.pi * n / _NFFT)
    k = np.arange(_NFREQ, dtype=f64)
    ang = 2.0 * np.pi * n[:, None] * k[None, :] / _NFFT
    basis = np.zeros((_NFFT, _F2PAD), f64)
    basis[:, :_NFREQ] = np.cos(ang) * win[:, None]
    basis[:, _NFREQ:2 * _NFREQ] = -np.sin(ang) * win[:, None]
    dmat = basis[:2 * _HOP]
    crow = basis[2 * _HOP:]

    hz2mel = lambda f: 2595.0 * np.log10(1.0 + f / 700.0)
    mel2hz = lambda m: 700.0 * (10.0 ** (m / 2595.0) - 1.0)
    freqs = np.linspace(0, _SR // 2, _NFREQ)
    mpts = mel2hz(np.linspace(hz2mel(0.0), hz2mel(_SR / 2.0), _NMELS + 2))
    fdiff = mpts[1:] - mpts[:-1]
    slopes = mpts[None, :] - freqs[:, None]
    fb = np.maximum(0.0, np.minimum(-slopes[:, :-2] / fdiff[:-1],
                                    slopes[:, 2:] / fdiff[1:]))
    fb2 = np.zeros((_F2PAD, _MELPAD), f64)
    fb2[:_NFREQ, :_NMELS] = fb
    fb2[_NFREQ:2 * _NFREQ, :_NMELS] = fb

    km = np.arange(_NMFCC, dtype=f64)[None, :]
    nm = np.arange(_NMELS, dtype=f64)[:, None]
    dct = np.cos(np.pi / _NMELS * (nm + 0.5) * km) * math.sqrt(2.0 / _NMELS)
    dct[:, 0] *= 1.0 / math.sqrt(2.0)
    dct_p = np.zeros((_MELPAD, _NMFCC), f64)
    dct_p[:_NMELS] = dct

    nd = (_DELTA_WIN - 1) // 2
    denom = nd * (nd + 1) * (2 * nd + 1) / 3.0
    dl = np.zeros((t_frames, t_frames), f64)
    ti = np.arange(t_frames)
    for j in range(-nd, nd + 1):
        np.add.at(dl, (ti, np.clip(ti + j, 0, t_frames - 1)), j / denom)

    scale = 0.5
    t_out = int(np.floor(t_frames * scale))
    src = np.maximum((np.arange(t_out) + 0.5) / scale - 0.5, 0.0)
    i0 = np.floor(src).astype(np.int64)
    i1 = np.minimum(i0 + 1, t_frames - 1)
    rs = np.zeros((t_out, t_frames), f64)
    np.add.at(rs, (np.arange(t_out), i0), 1.0 - (src - i0))
    np.add.at(rs, (np.arange(t_out), i1), src - i0)

    t_out_pad = max(8, _rup(t_out, 8))
    pstk = np.zeros((3 * t_out_pad, t_frames), f64)
    for g, m in enumerate((rs, rs @ dl, rs @ dl @ dl)):
        pstk[g * t_out_pad:g * t_out_pad + t_out] = m

    return (jnp.asarray(dmat, jnp.bfloat16), jnp.asarray(crow, jnp.float32),
            jnp.asarray(fb2, jnp.bfloat16), jnp.asarray(dct_p, jnp.float32),
            jnp.asarray(pstk, jnp.float32), t_out, t_out_pad)


_ITEMS = 4


def _make_spec_kernel(t_frames):
    def body(w_ref, rev_ref, dmat_ref, crow_ref, fb2_ref, mel_ref, max_ref):
        for it in range(_ITEMS):
            w = w_ref[it]
            w0 = jnp.concatenate([rev_ref[it], w[:t_frames - 1]],
                                 axis=0).astype(jnp.bfloat16)
            w1 = w.astype(jnp.bfloat16)
            col = jnp.concatenate(
                [w[1:, 0:1], w[t_frames - 1:, _HOP - 2:_HOP - 1]], axis=0)

            y = jnp.dot(w0, dmat_ref[0:_HOP],
                        preferred_element_type=jnp.float32)
            y = y + jnp.dot(w1, dmat_ref[_HOP:2 * _HOP],
                            preferred_element_type=jnp.float32)
            y = y + col * crow_ref[...]

            p = (y * y).astype(jnp.bfloat16)
            mel = jnp.dot(p, fb2_ref[...], preferred_element_type=jnp.float32)
            db = 10.0 * jnp.log10(jnp.maximum(mel, 1e-10))

            mel_ref[it] = db
            max_ref[it] = jnp.broadcast_to(jnp.max(db), (8, 128))
    return body


def _make_proj_kernel(t_out, t_out_pad):
    def body(max_ref, mel_ref, dct_ref, pstk_ref, out_ref):
        floor = jnp.max(max_ref[...]) - _TOPDB
        for it in range(_ITEMS):
            mel = jnp.maximum(mel_ref[it], floor)
            mfcc = jnp.dot(mel, dct_ref[...],
                           preferred_element_type=jnp.float32)
            acc = jnp.dot(pstk_ref[...], mfcc,
                          preferred_element_type=jnp.float32)
            out_ref[it] = jnp.concatenate(
                [acc[0:t_out],
                 acc[t_out_pad:t_out_pad + t_out],
                 acc[2 * t_out_pad:2 * t_out_pad + t_out]], axis=-1)
    return body


def kernel(wav, wav_lens, phn, phn_lens):
    del wav_lens, phn, phn_lens
    b = wav.shape[0]
    wav2d = wav.reshape(b, -1).astype(jnp.float32)
    L = wav2d.shape[1]
    assert L % _HOP == 0
    t = L // _HOP
    assert t % 8 == 0, "frame count must be a sublane multiple"

    dmat, crow, fb2, dct_p, pstk, t_out, t_out_pad = _consts(t)
    pr = pstk.shape[0]
    assert b % _ITEMS == 0
    n_g = b // _ITEMS

    wb = wav2d.reshape(b, t, _HOP)
    rev = wav2d[:, 1:_HOP + 1][:, ::-1].reshape(b, 1, _HOP)

    mel_db, bmax = pl.pallas_call(
        _make_spec_kernel(t),
        out_shape=(
            jax.ShapeDtypeStruct((b, t, _MELPAD), jnp.float32),
            jax.ShapeDtypeStruct((b, 8, 128), jnp.float32),
        ),
        grid=(n_g,),
        in_specs=[
            pl.BlockSpec((_ITEMS, t, _HOP), lambda i: (i, 0, 0)),
            pl.BlockSpec((_ITEMS, 1, _HOP), lambda i: (i, 0, 0)),
            pl.BlockSpec((2 * _HOP, _F2PAD), lambda i: (0, 0)),
            pl.BlockSpec((1, _F2PAD), lambda i: (0, 0)),
            pl.BlockSpec((_F2PAD, _MELPAD), lambda i: (0, 0)),
        ],
        out_specs=[
            pl.BlockSpec((_ITEMS, t, _MELPAD), lambda i: (i, 0, 0)),
            pl.BlockSpec((_ITEMS, 8, 128), lambda i: (i, 0, 0)),
        ],
        compiler_params=pltpu.CompilerParams(
            dimension_semantics=("parallel",),
            vmem_limit_bytes=96 * 1024 * 1024,
        ),
    )(wb, rev, dmat, crow, fb2)

    out = pl.pallas_call(
        _make_proj_kernel(t_out, t_out_pad),
        out_shape=jax.ShapeDtypeStruct((b, t_out, 3 * _NMFCC), jnp.float32),
        grid=(n_g,),
        in_specs=[
            pl.BlockSpec((b, 8, 128), lambda i: (0, 0, 0)),
            pl.BlockSpec((_ITEMS, t, _MELPAD), lambda i: (i, 0, 0)),
            pl.BlockSpec((_MELPAD, _NMFCC), lambda i: (0, 0)),
            pl.BlockSpec((pr, t), lambda i: (0, 0)),
        ],
        out_specs=pl.BlockSpec((_ITEMS, t_out, 3 * _NMFCC),
                               lambda i: (i, 0, 0)),
        compiler_params=pltpu.CompilerParams(
            dimension_semantics=("parallel",),
            vmem_limit_bytes=96 * 1024 * 1024,
        ),
    )(bmax, mel_db, dct_p, pstk)
    return [out]
